# baseline (speedup 1.0000x reference)
import functools
import math

import jax
import jax.numpy as jnp
from jax import lax
from jax.experimental import pallas as pl
from jax.experimental.pallas import tpu as pltpu
from jax.experimental.pallas import tpu_sc as plsc

SCALE = 32.0
ALPHA = 0.2
RBLK = 8


def _gather_targets(y32, score_flat, batch, num_cls):
    info = plsc.get_sparse_core_info()
    nw = info.num_cores * info.num_subcores
    bpw = batch // nw

    mesh = plsc.VectorSubcoreMesh(core_axis_name="c", subcore_axis_name="s")

    @functools.partial(
        pl.kernel,
        mesh=mesh,
        out_type=jax.ShapeDtypeStruct((batch,), jnp.float32),
        scratch_types=[
            pltpu.VMEM((bpw,), jnp.int32),
            pltpu.VMEM((bpw,), jnp.int32),
            pltpu.VMEM((bpw,), jnp.float32),
            pltpu.SemaphoreType.DMA,
        ],
    )
    def k(y_hbm, flat_hbm, out_hbm, y_v, idx_v, vals_v, sem):
        wid = lax.axis_index("s") * info.num_cores + lax.axis_index("c")
        base = wid * bpw
        pltpu.sync_copy(y_hbm.at[pl.ds(base, bpw)], y_v)
        for c in range(bpw // 16):
            rows = base + c * 16 + lax.iota(jnp.int32, 16)
            idx_v[pl.ds(c * 16, 16)] = y_v[pl.ds(c * 16, 16)] + rows * num_cls
        pltpu.async_copy(flat_hbm.at[idx_v], vals_v, sem).wait()
        pltpu.sync_copy(vals_v, out_hbm.at[pl.ds(base, bpw)])

    return k(y32, score_flat)


def _dense_loss(score, t_col, batch, num_cls):
    corr = math.exp(-SCALE * ALPHA) - 1.0
    parts = 4
    rpp = batch // parts
    grid = rpp // RBLK

    def body(*refs):
        t_refs = refs[:parts]
        score_refs = refs[parts : 2 * parts]
        out_refs = refs[2 * parts :]
        for t_ref, s_ref, o_ref in zip(t_refs, score_refs, out_refs):
            v = s_ref[...] * SCALE
            m = jnp.max(v, axis=1, keepdims=True)
            s = jnp.sum(jnp.exp(v - m), axis=1, keepdims=True)
            tt = t_ref[...] * SCALE
            o_ref[...] = m + jnp.log(s + jnp.exp(tt - m) * corr) - tt + SCALE * ALPHA

    in_specs = [pl.BlockSpec((RBLK, 1), lambda i: (i, 0)) for _ in range(parts)]
    in_specs += [
        pl.BlockSpec((RBLK, num_cls), lambda i, k=k: (i + k * grid, 0))
        for k in range(parts)
    ]
    t_parts = [t_col[k * rpp : (k + 1) * rpp] for k in range(parts)]
    outs = pl.pallas_call(
        body,
        grid=(grid,),
        in_specs=in_specs,
        out_specs=[pl.BlockSpec((RBLK, 1), lambda i: (i, 0))] * parts,
        out_shape=[jax.ShapeDtypeStruct((rpp, 1), jnp.float32)] * parts,
        compiler_params=pltpu.CompilerParams(
            dimension_semantics=("arbitrary",)
        ),
    )(*t_parts, *([score] * parts))
    return jnp.concatenate(outs, axis=0)


def kernel(score, y):
    batch, num_cls = score.shape
    y32 = jnp.asarray(y).reshape(-1).astype(jnp.int32)
    t = _gather_targets(y32, score.reshape(-1), batch, num_cls)
    out = _dense_loss(score, t.reshape(batch, 1), batch, num_cls)
    return out[:, 0]
